# baseline (speedup 1.0000x reference)
import functools

import jax
import jax.numpy as jnp
from jax import lax
from jax.experimental import pallas as pl
from jax.experimental.pallas import tpu as pltpu
from jax.experimental.pallas import tpu_sc as plsc

D = 1024
H = 8
HD = D // H
NS = 6
TOPK = 2
IMG = 224
P = 16
NP_ = (IMG // P) ** 2
PD = 3 * P * P
SL = 77
F32 = jnp.float32

_INTERPRET = False

BF16 = jnp.bfloat16


def _dot(a, b):
    return jnp.dot(a.astype(BF16), b.astype(BF16), preferred_element_type=F32)


def _dot_t(a, b):
    return lax.dot_general(a.astype(BF16), b.astype(BF16),
                           (((1,), (1,)), ((), ())),
                           preferred_element_type=F32)


def _fold8(a):
    a = a[:, :4] + a[:, 4:8]
    a = a[:, :2] + a[:, 2:4]
    return a[:, :1] + a[:, 1:2]


def _rowsum(x):
    r, n = x.shape
    if n <= 8:
        if n < 8:
            x = jnp.concatenate([x, jnp.zeros((r, 8 - n), x.dtype)], 1)
        return _fold8(x)
    nch = -(-n // 128)
    if n < nch * 128:
        x = jnp.concatenate([x, jnp.zeros((r, nch * 128 - n), x.dtype)], 1)
    acc = x[:, :128]
    for i in range(1, nch):
        acc = acc + x[:, i * 128:(i + 1) * 128]
    g = acc.reshape(r, 16, 8)
    a = g[:, 0, :]
    for v in range(1, 16):
        a = a + g[:, v, :]
    return _fold8(a)


def _softmax(x):
    e = jnp.exp(x - jnp.max(x, axis=-1, keepdims=True))
    return e / _rowsum(e)


def _ln(x, s, b):
    n = x.shape[-1]
    m = _rowsum(x) / n
    xc = x - m
    v = _rowsum(xc * xc) / n
    return xc / jnp.sqrt(v + 1e-5) * s[None, :] + b[None, :]


def _patch_kernel(p_ref, pw_ref, pb_ref, pos_ref, o_ref):
    x = p_ref[0]
    y = _dot(x, pw_ref[...])
    o_ref[0] = y + pb_ref[...][None, :] + pos_ref[0]


def _patch_embed(patches, pw, pb, pos):
    b = patches.shape[0]
    return pl.pallas_call(
        _patch_kernel,
        grid=(b,),
        in_specs=[
            pl.BlockSpec((1, NP_, PD), lambda i: (i, 0, 0)),
            pl.BlockSpec((PD, D), lambda i: (0, 0)),
            pl.BlockSpec((D,), lambda i: (0,)),
            pl.BlockSpec((1, NP_, D), lambda i: (0, 0, 0)),
        ],
        out_specs=pl.BlockSpec((1, NP_, D), lambda i: (i, 0, 0)),
        out_shape=jax.ShapeDtypeStruct((b, NP_, D), F32),
        interpret=_INTERPRET,
    )(patches, pw, pb, pos)


def _add_kernel(a_ref, b_ref, o_ref):
    o_ref[0] = a_ref[0] + b_ref[0]


def _add_pos(x, pos):
    b, s, d = x.shape
    return pl.pallas_call(
        _add_kernel,
        grid=(b,),
        in_specs=[
            pl.BlockSpec((1, s, d), lambda i: (i, 0, 0)),
            pl.BlockSpec((1, s, d), lambda i: (0, 0, 0)),
        ],
        out_specs=pl.BlockSpec((1, s, d), lambda i: (i, 0, 0)),
        out_shape=jax.ShapeDtypeStruct((b, s, d), F32),
        interpret=_INTERPRET,
    )(x, pos)


def _attn_kernel(x_ref, s_ref, b_ref, wq_ref, bq_ref, wk_ref, bk_ref,
                 wv_ref, bv_ref, wo_ref, bo_ref, o_ref):
    x = x_ref[0]
    xln = _ln(x, s_ref[...], b_ref[...])
    q = _dot(xln, wq_ref[...]) + bq_ref[...][None, :]
    k = _dot(xln, wk_ref[...]) + bk_ref[...][None, :]
    v = _dot(xln, wv_ref[...]) + bv_ref[...][None, :]
    scale = jnp.sqrt(jnp.float32(HD))
    outs = []
    for h in range(H):
        qh = q[:, h * HD:(h + 1) * HD]
        kh = k[:, h * HD:(h + 1) * HD]
        vh = v[:, h * HD:(h + 1) * HD]
        p = _softmax(_dot_t(qh, kh) / scale)
        outs.append(_dot(p, vh))
    o = jnp.concatenate(outs, axis=-1)
    o_ref[0] = x + _dot(o, wo_ref[...]) + bo_ref[...][None, :]


def _attention(x, p):
    b, s, d = x.shape
    wspec = pl.BlockSpec((D, D), lambda i: (0, 0))
    bspec = pl.BlockSpec((D,), lambda i: (0,))
    return pl.pallas_call(
        _attn_kernel,
        grid=(b,),
        in_specs=[
            pl.BlockSpec((1, s, d), lambda i: (i, 0, 0)),
            bspec, bspec,
            wspec, bspec, wspec, bspec, wspec, bspec, wspec, bspec,
        ],
        out_specs=pl.BlockSpec((1, s, d), lambda i: (i, 0, 0)),
        out_shape=jax.ShapeDtypeStruct((b, s, d), F32),
        interpret=_INTERPRET,
    )(x, p['ln1s'], p['ln1b'], p['wq'], p['bq'], p['wk'], p['bk'],
      p['wv'], p['bv'], p['wo'], p['bo'])


def _router_kernel(x_ref, s_ref, b_ref, rw_ref, rb_ref,
                   masks_ref, imp_ref, load_ref, loss_ref, *, ne, nb, s, use_ln):
    bidx = pl.program_id(0)
    x = x_ref[0]
    if use_ln:
        x = _ln(x, s_ref[...], b_ref[...])
    logits = _dot(x, rw_ref[...]) + rb_ref[...][None, :]
    probs = _softmax(logits)
    iota = lax.broadcasted_iota(jnp.int32, probs.shape, 1)
    big = jnp.int32(1 << 20)
    m1 = jnp.max(probs, axis=-1, keepdims=True)
    i1 = jnp.min(jnp.where(probs == m1, iota, big), axis=-1, keepdims=True)
    sel1 = iota == i1
    p2 = jnp.where(sel1, -jnp.inf, probs)
    m2 = jnp.max(p2, axis=-1, keepdims=True)
    i2 = jnp.min(jnp.where(p2 == m2, iota, big), axis=-1, keepdims=True)
    sel2 = iota == i2
    den = m1 + m2 + 1e-9
    masks = jnp.where(sel1, m1 / den, 0.0) + jnp.where(sel2, m2 / den, 0.0)
    masks_ref[0] = masks.astype(F32)

    @pl.when(bidx == 0)
    def _():
        imp_ref[...] = jnp.zeros_like(imp_ref)
        load_ref[...] = jnp.zeros_like(load_ref)

    denom = jnp.float32(nb * s)
    imp_ref[...] += jnp.sum(probs, axis=0, keepdims=True) / denom
    cnt = (sel1.astype(F32) + sel2.astype(F32))
    load_ref[...] += jnp.sum(cnt, axis=0, keepdims=True) / (denom * TOPK)

    @pl.when(bidx == nb - 1)
    def _():
        loss_ref[...] = ne * jnp.sum(imp_ref[...] * load_ref[...],
                                     keepdims=True)


def _router(x, lns, lnb, rw, rb, ne, use_ln):
    b, s, d = x.shape
    if lns is None:
        lns = jnp.ones((D,), F32)
        lnb = jnp.zeros((D,), F32)
    kfn = functools.partial(_router_kernel, ne=ne, nb=b, s=s, use_ln=use_ln)
    masks, imp, load, loss = pl.pallas_call(
        kfn,
        grid=(b,),
        in_specs=[
            pl.BlockSpec((1, s, d), lambda i: (i, 0, 0)),
            pl.BlockSpec((D,), lambda i: (0,)),
            pl.BlockSpec((D,), lambda i: (0,)),
            pl.BlockSpec((D, ne), lambda i: (0, 0)),
            pl.BlockSpec((ne,), lambda i: (0,)),
        ],
        out_specs=[
            pl.BlockSpec((1, s, ne), lambda i: (i, 0, 0)),
            pl.BlockSpec((1, ne), lambda i: (0, 0)),
            pl.BlockSpec((1, ne), lambda i: (0, 0)),
            pl.BlockSpec((1, 1), lambda i: (0, 0)),
        ],
        out_shape=[
            jax.ShapeDtypeStruct((b, s, ne), F32),
            jax.ShapeDtypeStruct((1, ne), F32),
            jax.ShapeDtypeStruct((1, ne), F32),
            jax.ShapeDtypeStruct((1, 1), F32),
        ],
        compiler_params=pltpu.CompilerParams(
            dimension_semantics=("arbitrary",)),
        interpret=_INTERPRET,
    )(x, lns, lnb, rw, rb)
    return masks, loss[0, 0]


_CH = 512


def _moe_kernel(x_ref, m_ref, w1_ref, b1_ref, w2_ref, b2_ref, o_ref, acc_ref,
                *, nk):
    e = pl.program_id(0)
    k = pl.program_id(1)

    @pl.when(jnp.logical_and(e == 0, k == 0))
    def _():
        o_ref[...] = jnp.zeros_like(o_ref)

    x = x_ref[...]
    h = _dot(x, w1_ref[0]) + b1_ref[0]
    h = jax.nn.gelu(h)
    part = _dot(h, w2_ref[0])

    @pl.when(k == 0)
    def _():
        acc_ref[...] = part

    @pl.when(k > 0)
    def _():
        acc_ref[...] += part

    @pl.when(k == nk - 1)
    def _():
        o_ref[...] += m_ref[0] * (acc_ref[...] + b2_ref[0])


def _moe(x_flat, masks_t, w1, b1, w2, b2):
    t = x_flat.shape[0]
    ne = w1.shape[0]
    nk = (4 * D) // _CH
    b1r = b1.reshape(ne * nk, 1, _CH)
    b2r = b2.reshape(ne, 1, D)
    kfn = functools.partial(_moe_kernel, nk=nk)
    return pl.pallas_call(
        kfn,
        grid=(ne, nk),
        in_specs=[
            pl.BlockSpec((t, D), lambda e, k: (0, 0)),
            pl.BlockSpec((1, t, 1), lambda e, k: (e, 0, 0)),
            pl.BlockSpec((1, D, _CH), lambda e, k: (e, 0, k)),
            pl.BlockSpec((1, 1, _CH), lambda e, k: (e * ((4 * D) // _CH) + k, 0, 0)),
            pl.BlockSpec((1, _CH, D), lambda e, k: (e, k, 0)),
            pl.BlockSpec((1, 1, D), lambda e, k: (e, 0, 0)),
        ],
        out_specs=pl.BlockSpec((t, D), lambda e, k: (0, 0)),
        out_shape=jax.ShapeDtypeStruct((t, D), F32),
        scratch_shapes=[pltpu.VMEM((t, D), F32)],
        compiler_params=pltpu.CompilerParams(
            dimension_semantics=("arbitrary", "arbitrary")),
        interpret=_INTERPRET,
    )(x_flat, masks_t, w1, b1r, w2, b2r)


def _final_kernel(eo_ref, s_ref, b_ref, cw_ref, cb_ref, mod_ref,
                  eo_out_ref, logits_ref):
    x = eo_ref[0]
    eo_ln = _ln(x, s_ref[...], b_ref[...])
    eo_out_ref[0] = eo_ln + mod_ref[...][None, :]
    feat = jnp.mean(eo_ln, axis=0, keepdims=True)
    logits_ref[0] = _dot(feat, cw_ref[...]) + cb_ref[...][None, :]


def _final(eo, lns, lnb, cw, cb, mod):
    b, s, d = eo.shape
    eo_mod, logits = pl.pallas_call(
        _final_kernel,
        grid=(b,),
        in_specs=[
            pl.BlockSpec((1, s, d), lambda i: (i, 0, 0)),
            pl.BlockSpec((D,), lambda i: (0,)),
            pl.BlockSpec((D,), lambda i: (0,)),
            pl.BlockSpec((D, D), lambda i: (0, 0)),
            pl.BlockSpec((D,), lambda i: (0,)),
            pl.BlockSpec((D,), lambda i: (0,)),
        ],
        out_specs=[
            pl.BlockSpec((1, s, d), lambda i: (i, 0, 0)),
            pl.BlockSpec((1, 1, D), lambda i: (i, 0, 0)),
        ],
        out_shape=[
            jax.ShapeDtypeStruct((b, s, d), F32),
            jax.ShapeDtypeStruct((b, 1, D), F32),
        ],
        interpret=_INTERPRET,
    )(eo, lns, lnb, cw, cb, mod)
    return eo_mod, logits.reshape(b, d)


def _mean_kernel(x_ref, o_ref):
    o_ref[0] = jnp.mean(x_ref[0], axis=0, keepdims=True)


def _mean_seq(x):
    b, s, d = x.shape
    out = pl.pallas_call(
        _mean_kernel,
        grid=(b,),
        in_specs=[pl.BlockSpec((1, s, d), lambda i: (i, 0, 0))],
        out_specs=pl.BlockSpec((1, 1, d), lambda i: (i, 0, 0)),
        out_shape=jax.ShapeDtypeStruct((b, 1, d), F32),
        interpret=_INTERPRET,
    )(x)
    return out.reshape(b, d)


def _embed_gather(table, ids_flat):
    info = plsc.get_sparse_core_info()
    nc, ns = info.num_cores, info.num_subcores
    nw = nc * ns
    bpad = ids_flat.shape[0]
    bpw = bpad // nw
    mesh = plsc.VectorSubcoreMesh(core_axis_name="c", subcore_axis_name="s")

    @functools.partial(
        pl.kernel, mesh=mesh,
        out_type=jax.ShapeDtypeStruct((bpad, D), F32),
        scratch_types=[
            pltpu.VMEM((bpw,), jnp.int32),
            pltpu.VMEM((bpw, D), F32),
            pltpu.SemaphoreType.DMA,
        ],
    )
    def k(table_hbm, idx_hbm, out_hbm, idx_v, rows_v, sem):
        wid = lax.axis_index("s") * nc + lax.axis_index("c")
        base = wid * bpw
        pltpu.sync_copy(idx_hbm.at[pl.ds(base, bpw)], idx_v)
        pltpu.async_copy(table_hbm.at[idx_v], rows_v, sem).wait()
        pltpu.sync_copy(rows_v, out_hbm.at[pl.ds(base, bpw)])

    return k(table, ids_flat)


def _text_embed(emb, input_ids):
    b, s = input_ids.shape
    nflat = b * s
    bpad = 2048
    ids = jnp.concatenate(
        [input_ids.reshape(nflat).astype(jnp.int32),
         jnp.zeros((bpad - nflat,), jnp.int32)])
    rows = _embed_gather(emb, ids)
    return rows[:nflat].reshape(b, s, D)


def _encoder(x, p, ne, mod):
    b, s, d = x.shape
    x2 = _attention(x, p)
    masks, rl = _router(x2, p['ln2s'], p['ln2b'], p['rw'], p['rb'], ne, True)
    masks_t = masks.reshape(b * s, ne).T[:, :, None]
    eo = _moe(x2.reshape(b * s, d), masks_t, p['ew1'], p['eb1'], p['ew2'], p['eb2'])
    eo_mod, logits = _final(eo.reshape(b, s, d), p['ln3s'], p['ln3b'],
                            p['cw'], p['cb'], mod)
    return eo_mod, logits, rl


def kernel(image, input_ids, img_params, txt_params, uni_params):
    b = image.shape[0]
    patches = image.reshape(b, 3, IMG // P, P, IMG // P, P)
    patches = patches.transpose(0, 2, 4, 3, 5, 1).reshape(b, NP_, PD)
    xi = _patch_embed(patches, img_params['pw'], img_params['pb'], img_params['pos'])
    img_eo, img_logits, img_rl = _encoder(xi, img_params, NS + 1,
                                          uni_params['mod'][0])

    xt = _text_embed(txt_params['emb'], input_ids)
    xt = _add_pos(xt, txt_params['pos'])
    txt_eo, txt_logits, txt_rl = _encoder(xt, txt_params, NS + 1,
                                          uni_params['mod'][1])

    comb = jnp.concatenate([img_eo, txt_eo], axis=1)
    ne = NS + 2
    masks, uni_rl = _router(comb, None, None, uni_params['rw'], uni_params['rb'],
                            ne, False)
    t = b * (NP_ + SL)
    masks_t = masks.reshape(t, ne).T[:, :, None]
    eo = _moe(comb.reshape(t, D), masks_t, uni_params['ew1'], uni_params['eb1'],
              uni_params['ew2'], uni_params['eb2'])
    fused = _mean_seq(eo.reshape(b, NP_ + SL, D))
    return fused, img_logits, txt_logits, img_rl + txt_rl + uni_rl
